# baseline (speedup 1.0000x reference)
import functools

import jax
import jax.numpy as jnp
from jax import lax
from jax.experimental import pallas as pl
from jax.experimental.pallas import tpu as pltpu
from jax.experimental.pallas import tpu_sc as plsc

N_NODES = 10000
N_EDGES = 320000
D_FEAT = 128
D_EDGE = 16

EH = N_EDGES // 2
NC = 2
NS = 16
NW = NC * NS
SLICES = (40960, 40960, 40960, 37120)
CR = 32
C = 2 * CR
ROWS_PER_TILE = 624
ROWS_TAIL = N_NODES - NS * ROWS_PER_TILE
LANES = 16
DP = D_FEAT // 2
DG2 = DP // LANES


def _pack_pairs(ab):
    lo = jax.lax.bitcast_convert_type(ab[:, :DP], jnp.uint16).astype(jnp.uint32)
    hi = jax.lax.bitcast_convert_type(ab[:, DP:], jnp.uint16).astype(jnp.uint32)
    return lo | (hi << 16)


def _gate_body(attr_a_ref, attr_b_ref, w_ref, out_ref):
    wb = w_ref[...].astype(jnp.bfloat16)

    def gate_half(attr):
        z = jnp.dot(attr.astype(jnp.bfloat16), wb,
                    preferred_element_type=jnp.float32)
        return _pack_pairs(jnp.tanh(z.astype(jnp.bfloat16)))

    out_ref[...] = jnp.concatenate(
        [gate_half(attr_a_ref[...]), gate_half(attr_b_ref[...])], axis=1)


def _compute_gate(attr_a, attr_b, W_edge, n_rows, be):
    nblk = n_rows // be
    return pl.pallas_call(
        _gate_body,
        grid=(nblk,),
        in_specs=[
            pl.BlockSpec((be, D_EDGE), lambda i: (i, 0)),
            pl.BlockSpec((be, D_EDGE), lambda i: (i, 0)),
            pl.BlockSpec((D_EDGE, D_FEAT), lambda i: (0, 0)),
        ],
        out_specs=pl.BlockSpec((be, D_FEAT), lambda i: (i, 0)),
        out_shape=jax.ShapeDtypeStruct((n_rows, D_FEAT), jnp.uint32),
    )(attr_a, attr_b, W_edge)


def _sc_body(n_rows, x_hbm, src_a, src_b, dst_a, dst_b, gate_hbm, init_hbm,
             out_hbm,
             src_v, dst_v, gate_v, rows_v,
             src_t, dst_t, gate_t, rows_t,
             agg_sh,
             semA0, semA1, semD0, semD1, semG0, semG1, semS0, semS1, semT):
    ewr = n_rows // NW
    nchunk = (ewr // CR) & ~1
    ng = nchunk // 2
    ctr = ewr - nchunk * CR
    semA = (semA0, semA1)
    semD = (semD0, semD1)
    semG = (semG0, semG1)
    semS = (semS0, semS1)
    cid = lax.axis_index("c")
    sid = lax.axis_index("s")
    wid = sid * NC + cid
    rbase = wid * ewr

    def start_a(i, b):
        base = pl.multiple_of(rbase + i * CR, 8)
        pltpu.async_copy(src_a.at[pl.ds(base, CR)],
                         src_v.at[b, pl.ds(0, CR)], semA[b])
        pltpu.async_copy(src_b.at[pl.ds(base, CR)],
                         src_v.at[b, pl.ds(CR, CR)], semA[b])
        pltpu.async_copy(gate_hbm.at[pl.ds(base, CR)], gate_v.at[b], semA[b])

    def wait_a(i, b):
        base = pl.multiple_of(rbase + i * CR, 8)
        pltpu.make_async_copy(src_a.at[pl.ds(base, CR)],
                              src_v.at[b, pl.ds(0, CR)], semA[b]).wait()
        pltpu.make_async_copy(src_b.at[pl.ds(base, CR)],
                              src_v.at[b, pl.ds(CR, CR)], semA[b]).wait()
        pltpu.make_async_copy(gate_hbm.at[pl.ds(base, CR)],
                              gate_v.at[b], semA[b]).wait()

    def start_d(i, b):
        base = pl.multiple_of(rbase + i * CR, 8)
        pltpu.async_copy(dst_a.at[pl.ds(base, CR)],
                         dst_v.at[b, pl.ds(0, CR)], semD[b])
        pltpu.async_copy(dst_b.at[pl.ds(base, CR)],
                         dst_v.at[b, pl.ds(CR, CR)], semD[b])

    def wait_d(i, b):
        base = pl.multiple_of(rbase + i * CR, 8)
        pltpu.make_async_copy(dst_a.at[pl.ds(base, CR)],
                              dst_v.at[b, pl.ds(0, CR)], semD[b]).wait()
        pltpu.make_async_copy(dst_b.at[pl.ds(base, CR)],
                              dst_v.at[b, pl.ds(CR, CR)], semD[b]).wait()

    def start_g(b):
        pltpu.async_copy(x_hbm.at[src_v.at[b]], rows_v.at[b], semG[b])

    def wait_g(b):
        pltpu.make_async_copy(x_hbm.at[src_v.at[b]], rows_v.at[b], semG[b]).wait()

    def start_s(b):
        pltpu.async_copy(rows_v.at[b], agg_sh.at[dst_v.at[b]], semS[b], add=True)

    def wait_s(b):
        pltpu.make_async_copy(rows_v.at[b], agg_sh.at[dst_v.at[b]], semS[b]).wait()

    hi_mask = jnp.full((LANES,), 0xFFFF0000, dtype=jnp.uint32)

    def bf16_split(w):
        lo = plsc.bitcast(w << 16, jnp.float32)
        hi = plsc.bitcast(w & hi_mask, jnp.float32)
        return lo, hi

    def multiply(buf_rows, buf_gate, n_half):
        def row(m, rc):
            for k in range(DG2):
                sl = pl.ds(k * LANES, LANES)
                sl_hi = pl.ds(DP + k * LANES, LANES)
                glo, ghi = bf16_split(buf_gate[m, sl])
                buf_rows[m, sl] = buf_rows[m, sl] * glo
                buf_rows[m, sl_hi] = buf_rows[m, sl_hi] * ghi
                glo2, ghi2 = bf16_split(buf_gate[m, sl_hi])
                buf_rows[n_half + m, sl] = buf_rows[n_half + m, sl] * glo2
                buf_rows[n_half + m, sl_hi] = buf_rows[n_half + m, sl_hi] * ghi2
            return rc
        lax.fori_loop(0, n_half, row, 0)

    r0 = pl.multiple_of(sid * ROWS_PER_TILE, 8)
    pltpu.sync_copy(init_hbm.at[cid, pl.ds(r0, ROWS_PER_TILE)],
                    agg_sh.at[pl.ds(r0, ROWS_PER_TILE)])

    @pl.when(sid == NS - 1)
    def _seed_tail():
        t0 = NS * ROWS_PER_TILE
        pltpu.sync_copy(init_hbm.at[cid, pl.ds(t0, ROWS_TAIL)],
                        agg_sh.at[pl.ds(t0, ROWS_TAIL)])

    plsc.subcore_barrier()

    start_a(0, 0)
    wait_a(0, 0)
    start_d(0, 0)
    start_g(0)
    start_a(1, 1)

    def pair(g, carry):
        for b in (0, 1):
            i = 2 * g + b
            o = 1 - b
            wait_g(b)
            def stage_next():
                wait_a(i + 1, o)
                if b == 0:
                    @pl.when(g >= 1)
                    def _():
                        wait_s(o)
                else:
                    wait_s(o)
                start_d(i + 1, o)
                start_g(o)
            if b == 0:
                stage_next()
            else:
                @pl.when(g < ng - 1)
                def _():
                    stage_next()
            multiply(rows_v.at[b], gate_v.at[b], CR)
            wait_d(i, b)
            start_s(b)
            @pl.when(g < ng - 1)
            def _():
                start_a(i + 2, b)
        return carry

    lax.fori_loop(0, ng, pair, 0)
    wait_s(0)
    wait_s(1)

    if ctr:
        tbase = pl.multiple_of(rbase + nchunk * CR, 8)
        pltpu.sync_copy(src_a.at[pl.ds(tbase, ctr)], src_t.at[pl.ds(0, ctr)])
        pltpu.sync_copy(src_b.at[pl.ds(tbase, ctr)], src_t.at[pl.ds(ctr, ctr)])
        pltpu.sync_copy(dst_a.at[pl.ds(tbase, ctr)], dst_t.at[pl.ds(0, ctr)])
        pltpu.sync_copy(dst_b.at[pl.ds(tbase, ctr)], dst_t.at[pl.ds(ctr, ctr)])
        pltpu.sync_copy(gate_hbm.at[pl.ds(tbase, ctr)], gate_t)
        pltpu.async_copy(x_hbm.at[src_t], rows_t, semT).wait()
        multiply(rows_t, gate_t, ctr)
        pltpu.sync_copy(rows_t, agg_sh.at[dst_t], add=True)

    plsc.subcore_barrier()
    pltpu.sync_copy(agg_sh.at[pl.ds(r0, ROWS_PER_TILE)],
                    out_hbm.at[cid, pl.ds(r0, ROWS_PER_TILE)])

    @pl.when(sid == NS - 1)
    def _write_tail():
        t0 = NS * ROWS_PER_TILE
        pltpu.sync_copy(agg_sh.at[pl.ds(t0, ROWS_TAIL)],
                        out_hbm.at[cid, pl.ds(t0, ROWS_TAIL)])


def _sc_scatter(n_rows, x, src_a, src_b, dst_a, dst_b, gate, init):
    ewr = n_rows // NW
    ctr = ewr - ((ewr // CR) & ~1) * CR
    ctr_alloc = max(ctr, 8)
    mesh = plsc.VectorSubcoreMesh(core_axis_name="c", subcore_axis_name="s")
    f = pl.kernel(
        functools.partial(_sc_body, n_rows),
        out_type=jax.ShapeDtypeStruct((NC, N_NODES, D_FEAT), jnp.float32),
        mesh=mesh,
        compiler_params=pltpu.CompilerParams(needs_layout_passes=False),
        scratch_types=[
            pltpu.VMEM((2, C), jnp.int32),
            pltpu.VMEM((2, C), jnp.int32),
            pltpu.VMEM((2, CR, D_FEAT), jnp.uint32),
            pltpu.VMEM((2, C, D_FEAT), jnp.float32),
            pltpu.VMEM((2 * ctr_alloc,), jnp.int32),
            pltpu.VMEM((2 * ctr_alloc,), jnp.int32),
            pltpu.VMEM((ctr_alloc, D_FEAT), jnp.uint32),
            pltpu.VMEM((2 * ctr_alloc, D_FEAT), jnp.float32),
            pltpu.VMEM_SHARED((N_NODES, D_FEAT), jnp.float32),
            pltpu.SemaphoreType.DMA,
            pltpu.SemaphoreType.DMA,
            pltpu.SemaphoreType.DMA,
            pltpu.SemaphoreType.DMA,
            pltpu.SemaphoreType.DMA,
            pltpu.SemaphoreType.DMA,
            pltpu.SemaphoreType.DMA,
            pltpu.SemaphoreType.DMA,
            pltpu.SemaphoreType.DMA,
        ],
    )
    return f(x, src_a, src_b, dst_a, dst_b, gate, init)


def _out_body(agg_ref, w_ref, out_ref):
    s = agg_ref[0] + agg_ref[1]
    out_ref[...] = jnp.dot(s, w_ref[...], preferred_element_type=jnp.float32)


def _project(partials, W_out):
    BR = 1000
    return pl.pallas_call(
        _out_body,
        grid=(N_NODES // BR,),
        in_specs=[
            pl.BlockSpec((NC, BR, D_FEAT), lambda i: (0, i, 0)),
            pl.BlockSpec((D_FEAT, D_FEAT), lambda i: (0, 0)),
        ],
        out_specs=pl.BlockSpec((BR, D_FEAT), lambda i: (i, 0)),
        out_shape=jax.ShapeDtypeStruct((N_NODES, D_FEAT), jnp.float32),
    )(partials, W_out)


def kernel(x, edge_index, edge_attr, W_edge, W_out):
    src = edge_index[0].astype(jnp.int32)
    dst = edge_index[1].astype(jnp.int32)

    gates = []
    args = []
    r0 = 0
    for h in SLICES:
        attr_a = edge_attr[r0:r0 + h]
        attr_b = edge_attr[EH + r0:EH + r0 + h]
        gates.append(_compute_gate(attr_a, attr_b, W_edge, h, h // 16))
        args.append((src[r0:r0 + h], src[EH + r0:EH + r0 + h],
                     dst[r0:r0 + h], dst[EH + r0:EH + r0 + h]))
        r0 += h

    p = jnp.zeros((NC, N_NODES, D_FEAT), jnp.float32)
    for h, gate, (sa, sb, da, db) in zip(SLICES, gates, args):
        p = _sc_scatter(h, x, sa, sb, da, db, gate, p)
    return _project(p, W_out)


# speedup vs baseline: 1.0158x; 1.0158x over previous
import functools

import jax
import jax.numpy as jnp
from jax import lax
from jax.experimental import pallas as pl
from jax.experimental.pallas import tpu as pltpu
from jax.experimental.pallas import tpu_sc as plsc

N_NODES = 10000
N_EDGES = 320000
D_FEAT = 128
D_EDGE = 16

EH = N_EDGES // 2
NC = 2
NS = 16
NW = NC * NS
SLICES = (53248, 53248, 53504)
CR = 32
C = 2 * CR
ROWS_PER_TILE = 624
ROWS_TAIL = N_NODES - NS * ROWS_PER_TILE
LANES = 16
DP = D_FEAT // 2
DG2 = DP // LANES


def _pack_pairs(ab):
    lo = jax.lax.bitcast_convert_type(ab[:, :DP], jnp.uint16).astype(jnp.uint32)
    hi = jax.lax.bitcast_convert_type(ab[:, DP:], jnp.uint16).astype(jnp.uint32)
    return lo | (hi << 16)


def _gate_body(attr_a_ref, attr_b_ref, w_ref, out_ref):
    wb = w_ref[...].astype(jnp.bfloat16)

    def gate_half(attr):
        z = jnp.dot(attr.astype(jnp.bfloat16), wb,
                    preferred_element_type=jnp.float32)
        return _pack_pairs(jnp.tanh(z.astype(jnp.bfloat16)))

    out_ref[...] = jnp.concatenate(
        [gate_half(attr_a_ref[...]), gate_half(attr_b_ref[...])], axis=1)


def _compute_gate(attr_a, attr_b, W_edge, n_rows, be):
    nblk = n_rows // be
    return pl.pallas_call(
        _gate_body,
        grid=(nblk,),
        in_specs=[
            pl.BlockSpec((be, D_EDGE), lambda i: (i, 0)),
            pl.BlockSpec((be, D_EDGE), lambda i: (i, 0)),
            pl.BlockSpec((D_EDGE, D_FEAT), lambda i: (0, 0)),
        ],
        out_specs=pl.BlockSpec((be, D_FEAT), lambda i: (i, 0)),
        out_shape=jax.ShapeDtypeStruct((n_rows, D_FEAT), jnp.uint32),
    )(attr_a, attr_b, W_edge)


def _sc_body(n_rows, x_hbm, src_a, src_b, dst_a, dst_b, gate_hbm, init_hbm,
             out_hbm,
             src_v, dst_v, gate_v, rows_v,
             src_t, dst_t, gate_t, rows_t,
             agg_sh,
             semA0, semA1, semD0, semD1, semG0, semG1, semS0, semS1, semT):
    ewr = n_rows // NW
    nchunk = (ewr // CR) & ~1
    ng = nchunk // 2
    ctr = ewr - nchunk * CR
    semA = (semA0, semA1)
    semD = (semD0, semD1)
    semG = (semG0, semG1)
    semS = (semS0, semS1)
    cid = lax.axis_index("c")
    sid = lax.axis_index("s")
    wid = sid * NC + cid
    rbase = wid * ewr

    def start_a(i, b):
        base = pl.multiple_of(rbase + i * CR, 8)
        pltpu.async_copy(src_a.at[pl.ds(base, CR)],
                         src_v.at[b, pl.ds(0, CR)], semA[b])
        pltpu.async_copy(src_b.at[pl.ds(base, CR)],
                         src_v.at[b, pl.ds(CR, CR)], semA[b])
        pltpu.async_copy(gate_hbm.at[pl.ds(base, CR)], gate_v.at[b], semA[b])

    def wait_a(i, b):
        base = pl.multiple_of(rbase + i * CR, 8)
        pltpu.make_async_copy(src_a.at[pl.ds(base, CR)],
                              src_v.at[b, pl.ds(0, CR)], semA[b]).wait()
        pltpu.make_async_copy(src_b.at[pl.ds(base, CR)],
                              src_v.at[b, pl.ds(CR, CR)], semA[b]).wait()
        pltpu.make_async_copy(gate_hbm.at[pl.ds(base, CR)],
                              gate_v.at[b], semA[b]).wait()

    def start_d(i, b):
        base = pl.multiple_of(rbase + i * CR, 8)
        pltpu.async_copy(dst_a.at[pl.ds(base, CR)],
                         dst_v.at[b, pl.ds(0, CR)], semD[b])
        pltpu.async_copy(dst_b.at[pl.ds(base, CR)],
                         dst_v.at[b, pl.ds(CR, CR)], semD[b])

    def wait_d(i, b):
        base = pl.multiple_of(rbase + i * CR, 8)
        pltpu.make_async_copy(dst_a.at[pl.ds(base, CR)],
                              dst_v.at[b, pl.ds(0, CR)], semD[b]).wait()
        pltpu.make_async_copy(dst_b.at[pl.ds(base, CR)],
                              dst_v.at[b, pl.ds(CR, CR)], semD[b]).wait()

    def start_g(b):
        pltpu.async_copy(x_hbm.at[src_v.at[b]], rows_v.at[b], semG[b])

    def wait_g(b):
        pltpu.make_async_copy(x_hbm.at[src_v.at[b]], rows_v.at[b], semG[b]).wait()

    def start_s(b):
        pltpu.async_copy(rows_v.at[b], agg_sh.at[dst_v.at[b]], semS[b], add=True)

    def wait_s(b):
        pltpu.make_async_copy(rows_v.at[b], agg_sh.at[dst_v.at[b]], semS[b]).wait()

    hi_mask = jnp.full((LANES,), 0xFFFF0000, dtype=jnp.uint32)

    def bf16_split(w):
        lo = plsc.bitcast(w << 16, jnp.float32)
        hi = plsc.bitcast(w & hi_mask, jnp.float32)
        return lo, hi

    def multiply(buf_rows, buf_gate, n_half):
        def row(m, rc):
            for k in range(DG2):
                sl = pl.ds(k * LANES, LANES)
                sl_hi = pl.ds(DP + k * LANES, LANES)
                glo, ghi = bf16_split(buf_gate[m, sl])
                buf_rows[m, sl] = buf_rows[m, sl] * glo
                buf_rows[m, sl_hi] = buf_rows[m, sl_hi] * ghi
                glo2, ghi2 = bf16_split(buf_gate[m, sl_hi])
                buf_rows[n_half + m, sl] = buf_rows[n_half + m, sl] * glo2
                buf_rows[n_half + m, sl_hi] = buf_rows[n_half + m, sl_hi] * ghi2
            return rc
        lax.fori_loop(0, n_half, row, 0)

    r0 = pl.multiple_of(sid * ROWS_PER_TILE, 8)
    pltpu.sync_copy(init_hbm.at[cid, pl.ds(r0, ROWS_PER_TILE)],
                    agg_sh.at[pl.ds(r0, ROWS_PER_TILE)])

    @pl.when(sid == NS - 1)
    def _seed_tail():
        t0 = NS * ROWS_PER_TILE
        pltpu.sync_copy(init_hbm.at[cid, pl.ds(t0, ROWS_TAIL)],
                        agg_sh.at[pl.ds(t0, ROWS_TAIL)])

    plsc.subcore_barrier()

    start_a(0, 0)
    wait_a(0, 0)
    start_d(0, 0)
    start_g(0)
    start_a(1, 1)

    def pair(g, carry):
        for b in (0, 1):
            i = 2 * g + b
            o = 1 - b
            wait_g(b)
            def stage_next():
                wait_a(i + 1, o)
                if b == 0:
                    @pl.when(g >= 1)
                    def _():
                        wait_s(o)
                else:
                    wait_s(o)
                start_d(i + 1, o)
                start_g(o)
            if b == 0:
                stage_next()
            else:
                @pl.when(g < ng - 1)
                def _():
                    stage_next()
            multiply(rows_v.at[b], gate_v.at[b], CR)
            wait_d(i, b)
            start_s(b)
            @pl.when(g < ng - 1)
            def _():
                start_a(i + 2, b)
        return carry

    lax.fori_loop(0, ng, pair, 0)
    wait_s(0)
    wait_s(1)

    if ctr:
        tbase = pl.multiple_of(rbase + nchunk * CR, 8)
        pltpu.sync_copy(src_a.at[pl.ds(tbase, ctr)], src_t.at[pl.ds(0, ctr)])
        pltpu.sync_copy(src_b.at[pl.ds(tbase, ctr)], src_t.at[pl.ds(ctr, ctr)])
        pltpu.sync_copy(dst_a.at[pl.ds(tbase, ctr)], dst_t.at[pl.ds(0, ctr)])
        pltpu.sync_copy(dst_b.at[pl.ds(tbase, ctr)], dst_t.at[pl.ds(ctr, ctr)])
        pltpu.sync_copy(gate_hbm.at[pl.ds(tbase, ctr)], gate_t)
        pltpu.async_copy(x_hbm.at[src_t], rows_t, semT).wait()
        multiply(rows_t, gate_t, ctr)
        pltpu.sync_copy(rows_t, agg_sh.at[dst_t], add=True)

    plsc.subcore_barrier()
    pltpu.sync_copy(agg_sh.at[pl.ds(r0, ROWS_PER_TILE)],
                    out_hbm.at[cid, pl.ds(r0, ROWS_PER_TILE)])

    @pl.when(sid == NS - 1)
    def _write_tail():
        t0 = NS * ROWS_PER_TILE
        pltpu.sync_copy(agg_sh.at[pl.ds(t0, ROWS_TAIL)],
                        out_hbm.at[cid, pl.ds(t0, ROWS_TAIL)])


def _sc_scatter(n_rows, x, src_a, src_b, dst_a, dst_b, gate, init):
    ewr = n_rows // NW
    ctr = ewr - ((ewr // CR) & ~1) * CR
    ctr_alloc = max(ctr, 8)
    mesh = plsc.VectorSubcoreMesh(core_axis_name="c", subcore_axis_name="s")
    f = pl.kernel(
        functools.partial(_sc_body, n_rows),
        out_type=jax.ShapeDtypeStruct((NC, N_NODES, D_FEAT), jnp.float32),
        mesh=mesh,
        compiler_params=pltpu.CompilerParams(needs_layout_passes=False),
        scratch_types=[
            pltpu.VMEM((2, C), jnp.int32),
            pltpu.VMEM((2, C), jnp.int32),
            pltpu.VMEM((2, CR, D_FEAT), jnp.uint32),
            pltpu.VMEM((2, C, D_FEAT), jnp.float32),
            pltpu.VMEM((2 * ctr_alloc,), jnp.int32),
            pltpu.VMEM((2 * ctr_alloc,), jnp.int32),
            pltpu.VMEM((ctr_alloc, D_FEAT), jnp.uint32),
            pltpu.VMEM((2 * ctr_alloc, D_FEAT), jnp.float32),
            pltpu.VMEM_SHARED((N_NODES, D_FEAT), jnp.float32),
            pltpu.SemaphoreType.DMA,
            pltpu.SemaphoreType.DMA,
            pltpu.SemaphoreType.DMA,
            pltpu.SemaphoreType.DMA,
            pltpu.SemaphoreType.DMA,
            pltpu.SemaphoreType.DMA,
            pltpu.SemaphoreType.DMA,
            pltpu.SemaphoreType.DMA,
            pltpu.SemaphoreType.DMA,
        ],
    )
    return f(x, src_a, src_b, dst_a, dst_b, gate, init)


def _out_body(agg_ref, w_ref, out_ref):
    s = agg_ref[0] + agg_ref[1]
    out_ref[...] = jnp.dot(s, w_ref[...], preferred_element_type=jnp.float32)


def _project(partials, W_out):
    BR = 1000
    return pl.pallas_call(
        _out_body,
        grid=(N_NODES // BR,),
        in_specs=[
            pl.BlockSpec((NC, BR, D_FEAT), lambda i: (0, i, 0)),
            pl.BlockSpec((D_FEAT, D_FEAT), lambda i: (0, 0)),
        ],
        out_specs=pl.BlockSpec((BR, D_FEAT), lambda i: (i, 0)),
        out_shape=jax.ShapeDtypeStruct((N_NODES, D_FEAT), jnp.float32),
    )(partials, W_out)


def kernel(x, edge_index, edge_attr, W_edge, W_out):
    src = edge_index[0].astype(jnp.int32)
    dst = edge_index[1].astype(jnp.int32)

    gates = []
    args = []
    r0 = 0
    for h in SLICES:
        attr_a = edge_attr[r0:r0 + h]
        attr_b = edge_attr[EH + r0:EH + r0 + h]
        gates.append(_compute_gate(attr_a, attr_b, W_edge, h, h // 16))
        args.append((src[r0:r0 + h], src[EH + r0:EH + r0 + h],
                     dst[r0:r0 + h], dst[EH + r0:EH + r0 + h]))
        r0 += h

    p = jnp.zeros((NC, N_NODES, D_FEAT), jnp.float32)
    for h, gate, (sa, sb, da, db) in zip(SLICES, gates, args):
        p = _sc_scatter(h, x, sa, sb, da, db, gate, p)
    return _project(p, W_out)
